# Initial kernel scaffold; baseline (speedup 1.0000x reference)
import jax
import jax.numpy as jnp
from jax.experimental import pallas as pl


def kernel(coords, features, W1, b1, g1, be1, W2, b2, g2, be2, W3, b3, g3, be3):
    raise NotImplementedError("write your pallas kernel here")


# speedup vs baseline: 2.8236x; 2.8236x over previous
import functools

import jax
import jax.numpy as jnp
from jax.experimental import pallas as pl
from jax.experimental.pallas import tpu as pltpu

B = 8
N = 4096
CF = 64
NFPS = 512
K = 32
RAD2 = 0.2 * 0.2
EPSBN = 1e-5
NSUB = 32
NLANE = 128
DPAD = 80
M_TOT = B * NFPS * K
TM = 512
TL = 2048

_HI = jax.lax.Precision.HIGHEST


def _dot(a, b, dims):
    return jax.lax.dot_general(a, b, (dims, ((), ())), precision=_HI,
                               preferred_element_type=jnp.float32)


def _fps_kernel(cr_ref, cent_ref):
    px = cr_ref[0, 0]
    py = cr_ref[0, 1]
    pz = cr_ref[0, 2]
    gidx = (jax.lax.broadcasted_iota(jnp.int32, (NSUB, NLANE), 0) * NLANE
            + jax.lax.broadcasted_iota(jnp.int32, (NSUB, NLANE), 1))

    def body(i, carry):
        d, f = carry
        sel = gidx == f
        cx = jnp.sum(jnp.where(sel, px, 0.0))
        cy = jnp.sum(jnp.where(sel, py, 0.0))
        cz = jnp.sum(jnp.where(sel, pz, 0.0))
        cent_ref[0, i, 0] = cx
        cent_ref[0, i, 1] = cy
        cent_ref[0, i, 2] = cz
        dx = px - cx
        dy = py - cy
        dz = pz - cz
        dist = dx * dx + dy * dy + dz * dz
        d = jnp.minimum(d, dist)
        dmax = jnp.max(d)
        f_new = jnp.min(jnp.where(d == dmax, gidx, N))
        return d, f_new

    d0 = jnp.full((NSUB, NLANE), 1e10, jnp.float32)
    jax.lax.fori_loop(0, NFPS, body, (d0, jnp.int32(0)))


def _run_fps(cr):
    return pl.pallas_call(
        _fps_kernel,
        grid=(B,),
        in_specs=[pl.BlockSpec((1, 3, NSUB, NLANE), lambda b: (b, 0, 0, 0))],
        out_specs=pl.BlockSpec((1, NFPS, 3), lambda b: (b, 0, 0),
                               memory_space=pltpu.SMEM),
        out_shape=jax.ShapeDtypeStruct((B, NFPS, 3), jnp.float32),
        compiler_params=pltpu.CompilerParams(
            dimension_semantics=("parallel",)),
    )(cr)


def _bq_kernel(cent_ref, cr_ref, tri_ref, idx_ref):
    cent = cent_ref[0]
    cx = cent[:, 0:1]
    cy = cent[:, 1:2]
    cz = cent[:, 2:3]
    a2 = cx * cx + cy * cy + cz * cz
    tri = tri_ref[...]

    def body(c, carry):
        acc, base = carry
        pc = cr_ref[0, :, c]
        pxc = pc[0:1, :]
        pyc = pc[1:2, :]
        pzc = pc[2:3, :]
        b2 = pxc * pxc + pyc * pyc + pzc * pzc
        ab = jax.lax.dot_general(cent, pc, (((1,), (0,)), ((), ())),
                                 preferred_element_type=jnp.float32)
        sqr = (a2 + b2) - 2.0 * ab
        maskf = jnp.where(sqr <= RAD2, 1.0, 0.0)
        local = _dot(maskf, tri, (((1,), (0,))))
        rank = base + local
        conts = [jnp.sum(jnp.where(rank <= float(j), 1.0, 0.0), axis=1,
                         keepdims=True) for j in range(K)]
        acc = acc + jnp.concatenate(conts, axis=1)
        base = base + jnp.sum(maskf, axis=1, keepdims=True)
        return acc, base

    acc0 = jnp.zeros((NFPS, K), jnp.float32)
    base0 = jnp.zeros((NFPS, 1), jnp.float32)
    acc, _ = jax.lax.fori_loop(0, NSUB, body, (acc0, base0))
    first = acc[:, 0:1]
    filled = jnp.where(acc == float(N), jnp.broadcast_to(first, acc.shape),
                       acc)
    idx_ref[0] = jnp.clip(filled, 0.0, float(N - 1)).astype(jnp.int32)


def _run_bq(cent, cr, tri):
    return pl.pallas_call(
        _bq_kernel,
        grid=(B,),
        in_specs=[
            pl.BlockSpec((1, NFPS, 3), lambda b: (b, 0, 0)),
            pl.BlockSpec((1, 3, NSUB, NLANE), lambda b: (b, 0, 0, 0)),
            pl.BlockSpec((NLANE, NLANE), lambda b: (0, 0)),
        ],
        out_specs=pl.BlockSpec((1, NFPS, K), lambda b: (b, 0, 0)),
        out_shape=jax.ShapeDtypeStruct((B, NFPS, K), jnp.int32),
        compiler_params=pltpu.CompilerParams(
            dimension_semantics=("parallel",)),
    )(cent, cr, tri)


def _gather_kernel(idx_ref, tab_ref, crep_ref, x_ref, g_ref):
    idxv = idx_ref[0, 0].reshape(TM, 1)
    iota = jax.lax.broadcasted_iota(jnp.int32, (TM, N), 1)
    oh = jnp.where(idxv == iota, 1.0, 0.0)
    x = _dot(oh, tab_ref[0], (((1,), (0,))))
    cr = crep_ref[0]
    xc = jnp.concatenate([x[:, 0:3] - cr, x[:, 3:]], axis=1)
    x_ref[...] = xc
    xa = jnp.concatenate([xc, jnp.ones((TM, 1), jnp.float32)], axis=1)
    g = _dot(xa, xa, (((0,), (0,))))

    @pl.when(pl.program_id(0) == 0)
    def _():
        g_ref[...] = jnp.zeros_like(g_ref)

    g_ref[...] += g


def _run_gather(idx3, tables, crep):
    steps = (B * NFPS * K) // TM
    tiles_per_b = (NFPS * K) // TM
    return pl.pallas_call(
        _gather_kernel,
        grid=(steps,),
        in_specs=[
            pl.BlockSpec((1, 1, TM), lambda s: (s, 0, 0)),
            pl.BlockSpec((1, N, DPAD), lambda s: (s // tiles_per_b, 0, 0)),
            pl.BlockSpec((1, TM, 3), lambda s: (s // tiles_per_b,
                                                s % tiles_per_b, 0)),
        ],
        out_specs=[
            pl.BlockSpec((TM, DPAD), lambda s: (s, 0)),
            pl.BlockSpec((DPAD + 1, DPAD + 1), lambda s: (0, 0)),
        ],
        out_shape=[
            jax.ShapeDtypeStruct((B * NFPS * K, DPAD), jnp.float32),
            jax.ShapeDtypeStruct((DPAD + 1, DPAD + 1), jnp.float32),
        ],
        compiler_params=pltpu.CompilerParams(
            dimension_semantics=("arbitrary",)),
    )(idx3, tables, crep)


def _stats_kernel(g_ref, wt_ref, ga_ref, be_ref, scale_ref, bias_ref, *,
                  din):
    G = g_ref[...]
    m = G[din:din + 1, 0:din] / M_TOT
    gx = G[0:din, 0:din] / M_TOT
    cov = gx - _dot(m, m, (((0,), (0,))))
    wt = wt_ref[...]
    mean_y = _dot(m, wt, (((1,), (0,))))
    covw = _dot(cov, wt, (((1,), (0,))))
    var_y = jnp.sum(wt * covw, axis=0, keepdims=True)
    scale = ga_ref[...] / jnp.sqrt(var_y + EPSBN)
    scale_ref[...] = scale
    bias_ref[...] = be_ref[...] - scale * mean_y


def _run_stats(g, wt, ga, be):
    din = wt.shape[0]
    dout = wt.shape[1]
    return pl.pallas_call(
        functools.partial(_stats_kernel, din=din),
        out_shape=[
            jax.ShapeDtypeStruct((1, dout), jnp.float32),
            jax.ShapeDtypeStruct((1, dout), jnp.float32),
        ],
    )(g, wt, ga, be)


def _layer_kernel(x_ref, wt_ref, s_ref, b_ref, v_ref, g_ref):
    y = _dot(x_ref[...], wt_ref[...], (((1,), (0,))))
    v = jnp.maximum(y * s_ref[...] + b_ref[...], 0.0)
    v_ref[...] = v
    va = jnp.concatenate([v, jnp.ones((TL, 1), jnp.float32)], axis=1)
    g = _dot(va, va, (((0,), (0,))))

    @pl.when(pl.program_id(0) == 0)
    def _():
        g_ref[...] = jnp.zeros_like(g_ref)

    g_ref[...] += g


def _run_layer(x, wt, scale, bias):
    steps = M_TOT // TL
    din = wt.shape[0]
    dout = wt.shape[1]
    return pl.pallas_call(
        _layer_kernel,
        grid=(steps,),
        in_specs=[
            pl.BlockSpec((TL, din), lambda s: (s, 0)),
            pl.BlockSpec((din, dout), lambda s: (0, 0)),
            pl.BlockSpec((1, dout), lambda s: (0, 0)),
            pl.BlockSpec((1, dout), lambda s: (0, 0)),
        ],
        out_specs=[
            pl.BlockSpec((TL, dout), lambda s: (s, 0)),
            pl.BlockSpec((dout + 1, dout + 1), lambda s: (0, 0)),
        ],
        out_shape=[
            jax.ShapeDtypeStruct((M_TOT, dout), jnp.float32),
            jax.ShapeDtypeStruct((dout + 1, dout + 1), jnp.float32),
        ],
        compiler_params=pltpu.CompilerParams(
            dimension_semantics=("arbitrary",)),
    )(x, wt, scale, bias)


def _layer3_kernel(x_ref, wt_ref, s_ref, b_ref, o_ref):
    y = _dot(x_ref[...], wt_ref[...], (((1,), (0,))))
    v = jnp.maximum(y * s_ref[...] + b_ref[...], 0.0)
    for g in range(TL // K):
        o_ref[g:g + 1, :] = jnp.max(v[g * K:(g + 1) * K, :], axis=0,
                                    keepdims=True)


def _run_layer3(x, wt, scale, bias):
    steps = M_TOT // TL
    din = wt.shape[0]
    dout = wt.shape[1]
    return pl.pallas_call(
        _layer3_kernel,
        grid=(steps,),
        in_specs=[
            pl.BlockSpec((TL, din), lambda s: (s, 0)),
            pl.BlockSpec((din, dout), lambda s: (0, 0)),
            pl.BlockSpec((1, dout), lambda s: (0, 0)),
            pl.BlockSpec((1, dout), lambda s: (0, 0)),
        ],
        out_specs=pl.BlockSpec((TL // K, dout), lambda s: (s, 0)),
        out_shape=jax.ShapeDtypeStruct((B * NFPS, dout), jnp.float32),
        compiler_params=pltpu.CompilerParams(
            dimension_semantics=("parallel",)),
    )(x, wt, scale, bias)


def kernel(coords, features, W1, b1, g1, be1, W2, b2, g2, be2,
           W3, b3, g3, be3):
    cr = coords.reshape(B, 3, NSUB, NLANE)
    cent = _run_fps(cr)

    tri = jnp.triu(jnp.ones((NLANE, NLANE), jnp.float32))
    idx = _run_bq(cent, cr, tri)

    tables = jnp.concatenate(
        [coords.transpose(0, 2, 1), features.transpose(0, 2, 1),
         jnp.zeros((B, N, DPAD - 3 - CF), jnp.float32)], axis=2)
    idx3 = idx.reshape((B * NFPS * K) // TM, 1, TM)
    crep = jnp.repeat(cent, K, axis=1)
    x, g0 = _run_gather(idx3, tables, crep)

    w1t = jnp.concatenate(
        [W1, jnp.zeros((W1.shape[0], DPAD - W1.shape[1]), jnp.float32)],
        axis=1).T
    s1, bi1 = _run_stats(g0, w1t, g1.reshape(1, -1), be1.reshape(1, -1))
    v1, gm1 = _run_layer(x, w1t, s1, bi1)

    w2t = W2.T
    s2, bi2 = _run_stats(gm1, w2t, g2.reshape(1, -1), be2.reshape(1, -1))
    v2, gm2 = _run_layer(v1, w2t, s2, bi2)

    w3t = W3.T
    s3, bi3 = _run_stats(gm2, w3t, g3.reshape(1, -1), be3.reshape(1, -1))
    pooled = _run_layer3(v2, w3t, s3, bi3)

    new_coords = cent.transpose(0, 2, 1)
    new_features = pooled.reshape(B, NFPS, -1).transpose(0, 2, 1)
    return (new_coords, new_features)


# speedup vs baseline: 3.2398x; 1.1474x over previous
import functools

import jax
import jax.numpy as jnp
from jax.experimental import pallas as pl
from jax.experimental.pallas import tpu as pltpu

B = 8
N = 4096
CF = 64
NFPS = 512
K = 32
RAD2 = 0.2 * 0.2
EPSBN = 1e-5
NSUB = 32
NLANE = 128
DPAD = 80
M_TOT = B * NFPS * K
TM = 512
TL = 2048

_HI = jax.lax.Precision.HIGHEST


def _dot(a, b, dims, precision=_HI):
    return jax.lax.dot_general(a, b, (dims, ((), ())), precision=precision,
                               preferred_element_type=jnp.float32)


BPC = 4


def _fps_kernel(cr_ref, cent_ref):
    pxs = [cr_ref[bi, 0] for bi in range(BPC)]
    pys = [cr_ref[bi, 1] for bi in range(BPC)]
    pzs = [cr_ref[bi, 2] for bi in range(BPC)]
    gidx = (jax.lax.broadcasted_iota(jnp.int32, (NSUB, NLANE), 0) * NLANE
            + jax.lax.broadcasted_iota(jnp.int32, (NSUB, NLANE), 1))

    def body(i, carry):
        out = []
        for bi in range(BPC):
            d, f = carry[bi]
            px, py, pz = pxs[bi], pys[bi], pzs[bi]
            sel = gidx == f
            cx = jnp.sum(jnp.where(sel, px, 0.0))
            cy = jnp.sum(jnp.where(sel, py, 0.0))
            cz = jnp.sum(jnp.where(sel, pz, 0.0))
            cent_ref[bi, 0, i] = cx
            cent_ref[bi, 1, i] = cy
            cent_ref[bi, 2, i] = cz
            dx = px - cx
            dy = py - cy
            dz = pz - cz
            dist = dx * dx + dy * dy + dz * dz
            d = jnp.minimum(d, dist)
            dmax = jnp.max(d)
            f_new = jnp.min(jnp.where(d == dmax, gidx, N))
            out.append((d, f_new))
        return tuple(out)

    d0 = jnp.full((NSUB, NLANE), 1e10, jnp.float32)
    jax.lax.fori_loop(0, NFPS,
                      body, tuple((d0, jnp.int32(0)) for _ in range(BPC)))


def _run_fps(cr):
    return pl.pallas_call(
        _fps_kernel,
        grid=(B // BPC,),
        in_specs=[pl.BlockSpec((BPC, 3, NSUB, NLANE),
                               lambda g: (g, 0, 0, 0))],
        out_specs=pl.BlockSpec((BPC, 3, NFPS), lambda g: (g, 0, 0),
                               memory_space=pltpu.SMEM),
        out_shape=jax.ShapeDtypeStruct((B, 3, NFPS), jnp.float32),
        compiler_params=pltpu.CompilerParams(
            dimension_semantics=("parallel",)),
    )(cr)


def _bq_kernel(cent_ref, cr_ref, tri_ref, idx_ref):
    cent = cent_ref[0]
    cx = cent[:, 0:1]
    cy = cent[:, 1:2]
    cz = cent[:, 2:3]
    a2 = cx * cx + cy * cy + cz * cz
    tri = tri_ref[...]

    def body(c, carry):
        acc, base = carry
        pc = cr_ref[0, :, c]
        pxc = pc[0:1, :]
        pyc = pc[1:2, :]
        pzc = pc[2:3, :]
        b2 = pxc * pxc + pyc * pyc + pzc * pzc
        ab = jax.lax.dot_general(cent, pc, (((1,), (0,)), ((), ())),
                                 preferred_element_type=jnp.float32)
        sqr = (a2 + b2) - 2.0 * ab
        maskf = jnp.where(sqr <= RAD2, 1.0, 0.0)
        local = _dot(maskf, tri, (((1,), (0,))), precision=None)
        rank = base + local
        conts = [jnp.sum(jnp.where(rank <= float(j), 1.0, 0.0), axis=1,
                         keepdims=True) for j in range(K)]
        acc = acc + jnp.concatenate(conts, axis=1)
        base = base + jnp.sum(maskf, axis=1, keepdims=True)
        return acc, base

    acc0 = jnp.zeros((NFPS, K), jnp.float32)
    base0 = jnp.zeros((NFPS, 1), jnp.float32)
    acc, _ = jax.lax.fori_loop(0, NSUB, body, (acc0, base0))
    first = acc[:, 0:1]
    filled = jnp.where(acc == float(N), jnp.broadcast_to(first, acc.shape),
                       acc)
    idx_ref[0] = jnp.clip(filled, 0.0, float(N - 1)).astype(jnp.int32)


def _run_bq(cent, cr, tri):
    return pl.pallas_call(
        _bq_kernel,
        grid=(B,),
        in_specs=[
            pl.BlockSpec((1, NFPS, 3), lambda b: (b, 0, 0)),
            pl.BlockSpec((1, 3, NSUB, NLANE), lambda b: (b, 0, 0, 0)),
            pl.BlockSpec((NLANE, NLANE), lambda b: (0, 0)),
        ],
        out_specs=pl.BlockSpec((1, NFPS, K), lambda b: (b, 0, 0)),
        out_shape=jax.ShapeDtypeStruct((B, NFPS, K), jnp.int32),
        compiler_params=pltpu.CompilerParams(
            dimension_semantics=("parallel",)),
    )(cent, cr, tri)


def _gather_kernel(idx_ref, tab_ref, crep_ref, x_ref, g_ref):
    idxv = idx_ref[0, 0].reshape(TM, 1)
    iota = jax.lax.broadcasted_iota(jnp.int32, (TM, N), 1)
    oh = jnp.where(idxv == iota, 1.0, 0.0)
    x = _dot(oh, tab_ref[0], (((1,), (0,))))
    cr = crep_ref[0]
    xc = jnp.concatenate([x[:, 0:3] - cr, x[:, 3:]], axis=1)
    x_ref[...] = xc
    xa = jnp.concatenate([xc, jnp.ones((TM, 1), jnp.float32)], axis=1)
    g = _dot(xa, xa, (((0,), (0,))), precision=None)

    @pl.when(pl.program_id(0) == 0)
    def _():
        g_ref[...] = jnp.zeros_like(g_ref)

    g_ref[...] += g


def _run_gather(idx3, tables, crep):
    steps = (B * NFPS * K) // TM
    tiles_per_b = (NFPS * K) // TM
    return pl.pallas_call(
        _gather_kernel,
        grid=(steps,),
        in_specs=[
            pl.BlockSpec((1, 1, TM), lambda s: (s, 0, 0)),
            pl.BlockSpec((1, N, DPAD), lambda s: (s // tiles_per_b, 0, 0)),
            pl.BlockSpec((1, TM, 3), lambda s: (s // tiles_per_b,
                                                s % tiles_per_b, 0)),
        ],
        out_specs=[
            pl.BlockSpec((TM, DPAD), lambda s: (s, 0)),
            pl.BlockSpec((DPAD + 1, DPAD + 1), lambda s: (0, 0)),
        ],
        out_shape=[
            jax.ShapeDtypeStruct((B * NFPS * K, DPAD), jnp.float32),
            jax.ShapeDtypeStruct((DPAD + 1, DPAD + 1), jnp.float32),
        ],
        compiler_params=pltpu.CompilerParams(
            dimension_semantics=("arbitrary",)),
    )(idx3, tables, crep)


def _stats_kernel(g_ref, wt_ref, ga_ref, be_ref, scale_ref, bias_ref, *,
                  din):
    G = g_ref[...]
    m = G[din:din + 1, 0:din] / M_TOT
    gx = G[0:din, 0:din] / M_TOT
    cov = gx - _dot(m, m, (((0,), (0,))))
    wt = wt_ref[...]
    mean_y = _dot(m, wt, (((1,), (0,))))
    covw = _dot(cov, wt, (((1,), (0,))))
    var_y = jnp.sum(wt * covw, axis=0, keepdims=True)
    scale = ga_ref[...] / jnp.sqrt(var_y + EPSBN)
    scale_ref[...] = scale
    bias_ref[...] = be_ref[...] - scale * mean_y


def _run_stats(g, wt, ga, be):
    din = wt.shape[0]
    dout = wt.shape[1]
    return pl.pallas_call(
        functools.partial(_stats_kernel, din=din),
        out_shape=[
            jax.ShapeDtypeStruct((1, dout), jnp.float32),
            jax.ShapeDtypeStruct((1, dout), jnp.float32),
        ],
    )(g, wt, ga, be)


def _layer_kernel(x_ref, wt_ref, s_ref, b_ref, v_ref, g_ref):
    y = _dot(x_ref[...], wt_ref[...], (((1,), (0,))))
    v = jnp.maximum(y * s_ref[...] + b_ref[...], 0.0)
    v_ref[...] = v
    va = jnp.concatenate([v, jnp.ones((TL, 1), jnp.float32)], axis=1)
    g = _dot(va, va, (((0,), (0,))), precision=None)

    @pl.when(pl.program_id(0) == 0)
    def _():
        g_ref[...] = jnp.zeros_like(g_ref)

    g_ref[...] += g


def _run_layer(x, wt, scale, bias):
    steps = M_TOT // TL
    din = wt.shape[0]
    dout = wt.shape[1]
    return pl.pallas_call(
        _layer_kernel,
        grid=(steps,),
        in_specs=[
            pl.BlockSpec((TL, din), lambda s: (s, 0)),
            pl.BlockSpec((din, dout), lambda s: (0, 0)),
            pl.BlockSpec((1, dout), lambda s: (0, 0)),
            pl.BlockSpec((1, dout), lambda s: (0, 0)),
        ],
        out_specs=[
            pl.BlockSpec((TL, dout), lambda s: (s, 0)),
            pl.BlockSpec((dout + 1, dout + 1), lambda s: (0, 0)),
        ],
        out_shape=[
            jax.ShapeDtypeStruct((M_TOT, dout), jnp.float32),
            jax.ShapeDtypeStruct((dout + 1, dout + 1), jnp.float32),
        ],
        compiler_params=pltpu.CompilerParams(
            dimension_semantics=("arbitrary",)),
    )(x, wt, scale, bias)


def _layer3_kernel(x_ref, wt_ref, s_ref, b_ref, o_ref):
    y = _dot(x_ref[...], wt_ref[...], (((1,), (0,))))
    v = jnp.maximum(y * s_ref[...] + b_ref[...], 0.0)
    for g in range(TL // K):
        o_ref[g:g + 1, :] = jnp.max(v[g * K:(g + 1) * K, :], axis=0,
                                    keepdims=True)


def _run_layer3(x, wt, scale, bias):
    steps = M_TOT // TL
    din = wt.shape[0]
    dout = wt.shape[1]
    return pl.pallas_call(
        _layer3_kernel,
        grid=(steps,),
        in_specs=[
            pl.BlockSpec((TL, din), lambda s: (s, 0)),
            pl.BlockSpec((din, dout), lambda s: (0, 0)),
            pl.BlockSpec((1, dout), lambda s: (0, 0)),
            pl.BlockSpec((1, dout), lambda s: (0, 0)),
        ],
        out_specs=pl.BlockSpec((TL // K, dout), lambda s: (s, 0)),
        out_shape=jax.ShapeDtypeStruct((B * NFPS, dout), jnp.float32),
        compiler_params=pltpu.CompilerParams(
            dimension_semantics=("parallel",)),
    )(x, wt, scale, bias)


def kernel(coords, features, W1, b1, g1, be1, W2, b2, g2, be2,
           W3, b3, g3, be3):
    cr = coords.reshape(B, 3, NSUB, NLANE)
    new_coords = _run_fps(cr)
    cent = new_coords.transpose(0, 2, 1)

    tri = jnp.triu(jnp.ones((NLANE, NLANE), jnp.float32))
    idx = _run_bq(cent, cr, tri)

    tables = jnp.concatenate(
        [coords.transpose(0, 2, 1), features.transpose(0, 2, 1),
         jnp.zeros((B, N, DPAD - 3 - CF), jnp.float32)], axis=2)
    idx3 = idx.reshape((B * NFPS * K) // TM, 1, TM)
    crep = jnp.repeat(cent, K, axis=1)
    x, g0 = _run_gather(idx3, tables, crep)

    w1t = jnp.concatenate(
        [W1, jnp.zeros((W1.shape[0], DPAD - W1.shape[1]), jnp.float32)],
        axis=1).T
    s1, bi1 = _run_stats(g0, w1t, g1.reshape(1, -1), be1.reshape(1, -1))
    v1, gm1 = _run_layer(x, w1t, s1, bi1)

    w2t = W2.T
    s2, bi2 = _run_stats(gm1, w2t, g2.reshape(1, -1), be2.reshape(1, -1))
    v2, gm2 = _run_layer(v1, w2t, s2, bi2)

    w3t = W3.T
    s3, bi3 = _run_stats(gm2, w3t, g3.reshape(1, -1), be3.reshape(1, -1))
    pooled = _run_layer3(v2, w3t, s3, bi3)

    new_features = pooled.reshape(B, NFPS, -1).transpose(0, 2, 1)
    return (new_coords, new_features)


# speedup vs baseline: 5.2221x; 1.6118x over previous
import functools

import jax
import jax.numpy as jnp
from jax import lax
from jax.experimental import pallas as pl
from jax.experimental.pallas import tpu as pltpu
from jax.experimental.pallas import tpu_sc as plsc

B = 8
N = 4096
CF = 64
NFPS = 512
K = 32
RAD2 = 0.2 * 0.2
EPSBN = 1e-5
NSUB = 32
NLANE = 128
DPAD = 80
M_TOT = B * NFPS * K
TM = 512
TL = 2048

_HI = jax.lax.Precision.HIGHEST


def _dot(a, b, dims, precision=_HI):
    return jax.lax.dot_general(a, b, (dims, ((), ())), precision=precision,
                               preferred_element_type=jnp.float32)


BPC = 4


def _fps_kernel(cr_ref, cent_ref):
    pxs = [cr_ref[bi, 0] for bi in range(BPC)]
    pys = [cr_ref[bi, 1] for bi in range(BPC)]
    pzs = [cr_ref[bi, 2] for bi in range(BPC)]
    gidx = (jax.lax.broadcasted_iota(jnp.int32, (NSUB, NLANE), 0) * NLANE
            + jax.lax.broadcasted_iota(jnp.int32, (NSUB, NLANE), 1))

    def body(i, carry):
        out = []
        for bi in range(BPC):
            d, f = carry[bi]
            px, py, pz = pxs[bi], pys[bi], pzs[bi]
            sel = gidx == f
            cx = jnp.sum(jnp.where(sel, px, 0.0))
            cy = jnp.sum(jnp.where(sel, py, 0.0))
            cz = jnp.sum(jnp.where(sel, pz, 0.0))
            cent_ref[bi, 0, i] = cx
            cent_ref[bi, 1, i] = cy
            cent_ref[bi, 2, i] = cz
            dx = px - cx
            dy = py - cy
            dz = pz - cz
            dist = dx * dx + dy * dy + dz * dz
            d = jnp.minimum(d, dist)
            dmax = jnp.max(d)
            f_new = jnp.min(jnp.where(d == dmax, gidx, N))
            out.append((d, f_new))
        return tuple(out)

    d0 = jnp.full((NSUB, NLANE), 1e10, jnp.float32)
    jax.lax.fori_loop(0, NFPS,
                      body, tuple((d0, jnp.int32(0)) for _ in range(BPC)))


def _run_fps(cr):
    return pl.pallas_call(
        _fps_kernel,
        grid=(B // BPC,),
        in_specs=[pl.BlockSpec((BPC, 3, NSUB, NLANE),
                               lambda g: (g, 0, 0, 0))],
        out_specs=pl.BlockSpec((BPC, 3, NFPS), lambda g: (g, 0, 0),
                               memory_space=pltpu.SMEM),
        out_shape=jax.ShapeDtypeStruct((B, 3, NFPS), jnp.float32),
        compiler_params=pltpu.CompilerParams(
            dimension_semantics=("parallel",)),
    )(cr)


def _bq_kernel(cent_ref, cr_ref, tri_ref, idx_ref):
    cent = cent_ref[0]
    cx = cent[:, 0:1]
    cy = cent[:, 1:2]
    cz = cent[:, 2:3]
    a2 = cx * cx + cy * cy + cz * cz
    tri = tri_ref[...]

    def body(c, carry):
        acc, base = carry
        pc = cr_ref[0, :, c]
        pxc = pc[0:1, :]
        pyc = pc[1:2, :]
        pzc = pc[2:3, :]
        b2 = pxc * pxc + pyc * pyc + pzc * pzc
        ab = jax.lax.dot_general(cent, pc, (((1,), (0,)), ((), ())),
                                 preferred_element_type=jnp.float32)
        sqr = (a2 + b2) - 2.0 * ab
        maskf = jnp.where(sqr <= RAD2, 1.0, 0.0)
        local = _dot(maskf, tri, (((1,), (0,))), precision=None)
        rank = base + local
        conts = [jnp.sum(jnp.where(rank <= float(j), 1.0, 0.0), axis=1,
                         keepdims=True) for j in range(K)]
        acc = acc + jnp.concatenate(conts, axis=1)
        base = base + jnp.sum(maskf, axis=1, keepdims=True)
        return acc, base

    acc0 = jnp.zeros((NFPS, K), jnp.float32)
    base0 = jnp.zeros((NFPS, 1), jnp.float32)
    acc, _ = jax.lax.fori_loop(0, NSUB, body, (acc0, base0))
    first = acc[:, 0:1]
    filled = jnp.where(acc == float(N), jnp.broadcast_to(first, acc.shape),
                       acc)
    local = jnp.clip(filled, 0.0, float(N - 1)).astype(jnp.int32)
    idx_ref[0] = local + pl.program_id(0) * N


def _run_bq(cent, cr, tri):
    return pl.pallas_call(
        _bq_kernel,
        grid=(B,),
        in_specs=[
            pl.BlockSpec((1, NFPS, 3), lambda b: (b, 0, 0)),
            pl.BlockSpec((1, 3, NSUB, NLANE), lambda b: (b, 0, 0, 0)),
            pl.BlockSpec((NLANE, NLANE), lambda b: (0, 0)),
        ],
        out_specs=pl.BlockSpec((1, NFPS, K), lambda b: (b, 0, 0)),
        out_shape=jax.ShapeDtypeStruct((B, NFPS, K), jnp.int32),
        compiler_params=pltpu.CompilerParams(
            dimension_semantics=("parallel",)),
    )(cent, cr, tri)


NW = 32
RPW = M_TOT // NW
CH = 128
NCH = RPW // CH


def _sc_gather_body(tab_ref, idx_ref, out_ref, idx_v, buf, sem):
    wid = lax.axis_index("s") * 2 + lax.axis_index("c")
    base = wid * RPW
    pltpu.sync_copy(idx_ref.at[pl.ds(base, RPW)], idx_v)

    def body(j, carry):
        off = pl.multiple_of(j * CH, CH)
        pltpu.async_copy(tab_ref.at[idx_v.at[pl.ds(off, CH)]], buf,
                         sem).wait()
        pltpu.sync_copy(buf, out_ref.at[pl.ds(base + off, CH)])
        return carry

    lax.fori_loop(0, NCH, body, 0)


def _run_sc_gather(table_all, idx_flat):
    mesh = plsc.VectorSubcoreMesh(core_axis_name="c", subcore_axis_name="s")
    k = functools.partial(
        pl.kernel,
        mesh=mesh,
        compiler_params=pltpu.CompilerParams(use_tc_tiling_on_sc=False),
        out_type=jax.ShapeDtypeStruct((M_TOT, DPAD), jnp.float32),
        scratch_types=[
            pltpu.VMEM((RPW,), jnp.int32),
            pltpu.VMEM((CH, DPAD), jnp.float32),
            pltpu.SemaphoreType.DMA,
        ],
    )(_sc_gather_body)
    return k(table_all, idx_flat)


def _gram_kernel(x_ref, c_ref, g_ref):
    x = x_ref[...]
    xc = jnp.concatenate([x[:, 0:3] - c_ref[...], x[:, 3:]], axis=1)
    xa = jnp.concatenate([xc, jnp.ones((TL, 1), jnp.float32)], axis=1)
    g = _dot(xa, xa, (((0,), (0,))), precision=None)

    @pl.when(pl.program_id(0) == 0)
    def _():
        g_ref[...] = jnp.zeros_like(g_ref)

    g_ref[...] += g


def _run_gram(x, crepf):
    steps = M_TOT // TL
    return pl.pallas_call(
        _gram_kernel,
        grid=(steps,),
        in_specs=[
            pl.BlockSpec((TL, DPAD), lambda s: (s, 0)),
            pl.BlockSpec((TL, 3), lambda s: (s, 0)),
        ],
        out_specs=pl.BlockSpec((DPAD + 1, DPAD + 1), lambda s: (0, 0)),
        out_shape=jax.ShapeDtypeStruct((DPAD + 1, DPAD + 1), jnp.float32),
        compiler_params=pltpu.CompilerParams(
            dimension_semantics=("arbitrary",)),
    )(x, crepf)


def _stats_kernel(g_ref, wt_ref, ga_ref, be_ref, scale_ref, bias_ref, *,
                  din):
    G = g_ref[...]
    m = G[din:din + 1, 0:din] / M_TOT
    gx = G[0:din, 0:din] / M_TOT
    cov = gx - _dot(m, m, (((0,), (0,))))
    wt = wt_ref[...]
    mean_y = _dot(m, wt, (((1,), (0,))))
    covw = _dot(cov, wt, (((1,), (0,))))
    var_y = jnp.sum(wt * covw, axis=0, keepdims=True)
    scale = ga_ref[...] / jnp.sqrt(var_y + EPSBN)
    scale_ref[...] = scale
    bias_ref[...] = be_ref[...] - scale * mean_y


def _run_stats(g, wt, ga, be):
    din = wt.shape[0]
    dout = wt.shape[1]
    return pl.pallas_call(
        functools.partial(_stats_kernel, din=din),
        out_shape=[
            jax.ShapeDtypeStruct((1, dout), jnp.float32),
            jax.ShapeDtypeStruct((1, dout), jnp.float32),
        ],
    )(g, wt, ga, be)


def _layer1_kernel(x_ref, c_ref, wt_ref, s_ref, b_ref, v_ref, g_ref):
    x = x_ref[...]
    xc = jnp.concatenate([x[:, 0:3] - c_ref[...], x[:, 3:]], axis=1)
    y = _dot(xc, wt_ref[...], (((1,), (0,))))
    v = jnp.maximum(y * s_ref[...] + b_ref[...], 0.0)
    v_ref[...] = v
    va = jnp.concatenate([v, jnp.ones((TL, 1), jnp.float32)], axis=1)
    g = _dot(va, va, (((0,), (0,))), precision=None)

    @pl.when(pl.program_id(0) == 0)
    def _():
        g_ref[...] = jnp.zeros_like(g_ref)

    g_ref[...] += g


def _run_layer1(x, crepf, wt, scale, bias):
    steps = M_TOT // TL
    din = wt.shape[0]
    dout = wt.shape[1]
    return pl.pallas_call(
        _layer1_kernel,
        grid=(steps,),
        in_specs=[
            pl.BlockSpec((TL, din), lambda s: (s, 0)),
            pl.BlockSpec((TL, 3), lambda s: (s, 0)),
            pl.BlockSpec((din, dout), lambda s: (0, 0)),
            pl.BlockSpec((1, dout), lambda s: (0, 0)),
            pl.BlockSpec((1, dout), lambda s: (0, 0)),
        ],
        out_specs=[
            pl.BlockSpec((TL, dout), lambda s: (s, 0)),
            pl.BlockSpec((dout + 1, dout + 1), lambda s: (0, 0)),
        ],
        out_shape=[
            jax.ShapeDtypeStruct((M_TOT, dout), jnp.float32),
            jax.ShapeDtypeStruct((dout + 1, dout + 1), jnp.float32),
        ],
        compiler_params=pltpu.CompilerParams(
            dimension_semantics=("arbitrary",)),
    )(x, crepf, wt, scale, bias)


def _layer_kernel(x_ref, wt_ref, s_ref, b_ref, v_ref, g_ref):
    y = _dot(x_ref[...], wt_ref[...], (((1,), (0,))))
    v = jnp.maximum(y * s_ref[...] + b_ref[...], 0.0)
    v_ref[...] = v
    va = jnp.concatenate([v, jnp.ones((TL, 1), jnp.float32)], axis=1)
    g = _dot(va, va, (((0,), (0,))), precision=None)

    @pl.when(pl.program_id(0) == 0)
    def _():
        g_ref[...] = jnp.zeros_like(g_ref)

    g_ref[...] += g


def _run_layer(x, wt, scale, bias):
    steps = M_TOT // TL
    din = wt.shape[0]
    dout = wt.shape[1]
    return pl.pallas_call(
        _layer_kernel,
        grid=(steps,),
        in_specs=[
            pl.BlockSpec((TL, din), lambda s: (s, 0)),
            pl.BlockSpec((din, dout), lambda s: (0, 0)),
            pl.BlockSpec((1, dout), lambda s: (0, 0)),
            pl.BlockSpec((1, dout), lambda s: (0, 0)),
        ],
        out_specs=[
            pl.BlockSpec((TL, dout), lambda s: (s, 0)),
            pl.BlockSpec((dout + 1, dout + 1), lambda s: (0, 0)),
        ],
        out_shape=[
            jax.ShapeDtypeStruct((M_TOT, dout), jnp.float32),
            jax.ShapeDtypeStruct((dout + 1, dout + 1), jnp.float32),
        ],
        compiler_params=pltpu.CompilerParams(
            dimension_semantics=("arbitrary",)),
    )(x, wt, scale, bias)


def _layer3_kernel(x_ref, wt_ref, s_ref, b_ref, o_ref):
    y = _dot(x_ref[...], wt_ref[...], (((1,), (0,))))
    v = jnp.maximum(y * s_ref[...] + b_ref[...], 0.0)
    for g in range(TL // K):
        o_ref[g:g + 1, :] = jnp.max(v[g * K:(g + 1) * K, :], axis=0,
                                    keepdims=True)


def _run_layer3(x, wt, scale, bias):
    steps = M_TOT // TL
    din = wt.shape[0]
    dout = wt.shape[1]
    return pl.pallas_call(
        _layer3_kernel,
        grid=(steps,),
        in_specs=[
            pl.BlockSpec((TL, din), lambda s: (s, 0)),
            pl.BlockSpec((din, dout), lambda s: (0, 0)),
            pl.BlockSpec((1, dout), lambda s: (0, 0)),
            pl.BlockSpec((1, dout), lambda s: (0, 0)),
        ],
        out_specs=pl.BlockSpec((TL // K, dout), lambda s: (s, 0)),
        out_shape=jax.ShapeDtypeStruct((B * NFPS, dout), jnp.float32),
        compiler_params=pltpu.CompilerParams(
            dimension_semantics=("parallel",)),
    )(x, wt, scale, bias)


def kernel(coords, features, W1, b1, g1, be1, W2, b2, g2, be2,
           W3, b3, g3, be3):
    cr = coords.reshape(B, 3, NSUB, NLANE)
    new_coords = _run_fps(cr)
    cent = new_coords.transpose(0, 2, 1)

    tri = jnp.triu(jnp.ones((NLANE, NLANE), jnp.float32))
    idx = _run_bq(cent, cr, tri)

    table_all = jnp.concatenate(
        [coords.transpose(0, 2, 1), features.transpose(0, 2, 1),
         jnp.zeros((B, N, DPAD - 3 - CF), jnp.float32)],
        axis=2).reshape(B * N, DPAD)
    x = _run_sc_gather(table_all, idx.reshape(-1))
    crepf = jnp.repeat(cent, K, axis=1).reshape(M_TOT, 3)
    g0 = _run_gram(x, crepf)

    w1t = jnp.concatenate(
        [W1, jnp.zeros((W1.shape[0], DPAD - W1.shape[1]), jnp.float32)],
        axis=1).T
    s1, bi1 = _run_stats(g0, w1t, g1.reshape(1, -1), be1.reshape(1, -1))
    v1, gm1 = _run_layer1(x, crepf, w1t, s1, bi1)

    w2t = W2.T
    s2, bi2 = _run_stats(gm1, w2t, g2.reshape(1, -1), be2.reshape(1, -1))
    v2, gm2 = _run_layer(v1, w2t, s2, bi2)

    w3t = W3.T
    s3, bi3 = _run_stats(gm2, w3t, g3.reshape(1, -1), be3.reshape(1, -1))
    pooled = _run_layer3(v2, w3t, s3, bi3)

    new_features = pooled.reshape(B, NFPS, -1).transpose(0, 2, 1)
    return (new_coords, new_features)


# speedup vs baseline: 5.8910x; 1.1281x over previous
import functools

import jax
import jax.numpy as jnp
from jax import lax
from jax.experimental import pallas as pl
from jax.experimental.pallas import tpu as pltpu
from jax.experimental.pallas import tpu_sc as plsc

B = 8
N = 4096
CF = 64
NFPS = 512
K = 32
RAD2 = 0.2 * 0.2
EPSBN = 1e-5
NSUB = 32
NLANE = 128
DPAD = 80
M_TOT = B * NFPS * K
TM = 512
TL = 2048

_HI = jax.lax.Precision.HIGHEST


def _dot(a, b, dims, precision=_HI):
    return jax.lax.dot_general(a, b, (dims, ((), ())), precision=precision,
                               preferred_element_type=jnp.float32)


BPC = 4


def _fps_kernel(cr_ref, cent_ref):
    pxs = [cr_ref[bi, 0] for bi in range(BPC)]
    pys = [cr_ref[bi, 1] for bi in range(BPC)]
    pzs = [cr_ref[bi, 2] for bi in range(BPC)]
    gidx = (jax.lax.broadcasted_iota(jnp.int32, (NSUB, NLANE), 0) * NLANE
            + jax.lax.broadcasted_iota(jnp.int32, (NSUB, NLANE), 1))

    def body(i, carry):
        out = []
        for bi in range(BPC):
            d, f = carry[bi]
            px, py, pz = pxs[bi], pys[bi], pzs[bi]
            sel = gidx == f
            cx = jnp.sum(jnp.where(sel, px, 0.0), keepdims=True)
            cy = jnp.sum(jnp.where(sel, py, 0.0), keepdims=True)
            cz = jnp.sum(jnp.where(sel, pz, 0.0), keepdims=True)
            cent_ref[bi, pl.ds(i, 1), :] = jnp.concatenate(
                [cx, cy, cz], axis=1)
            dx = px - cx
            dy = py - cy
            dz = pz - cz
            dist = (dx * dx + dz * dz) + dy * dy
            d = jnp.minimum(d, dist)
            dmax = jnp.max(d, keepdims=True)
            f_new = jnp.min(jnp.where(d == dmax, gidx, N), keepdims=True)
            out.append((d, f_new))
        return tuple(out)

    d0 = jnp.full((NSUB, NLANE), 1e10, jnp.float32)
    f0 = jnp.zeros((1, 1), jnp.int32)
    jax.lax.fori_loop(0, NFPS, body, tuple((d0, f0) for _ in range(BPC)))


def _run_fps(cr):
    return pl.pallas_call(
        _fps_kernel,
        grid=(B // BPC,),
        in_specs=[pl.BlockSpec((BPC, 3, NSUB, NLANE),
                               lambda g: (g, 0, 0, 0))],
        out_specs=pl.BlockSpec((BPC, NFPS, 3), lambda g: (g, 0, 0)),
        out_shape=jax.ShapeDtypeStruct((B, NFPS, 3), jnp.float32),
        compiler_params=pltpu.CompilerParams(
            dimension_semantics=("parallel",)),
    )(cr)


BQW = 512
BQC = N // BQW


def _bq_kernel(cent_ref, cr_ref, tri_ref, idx_ref):
    cent = cent_ref[0]
    cx = cent[:, 0:1]
    cy = cent[:, 1:2]
    cz = cent[:, 2:3]
    a2 = cx * cx + cy * cy + cz * cz
    tri = tri_ref[...]

    def body(c, carry):
        acc, base = carry
        pc = cr_ref[0, :, c]
        pxc = pc[0:1, :]
        pyc = pc[1:2, :]
        pzc = pc[2:3, :]
        b2 = pxc * pxc + pyc * pyc + pzc * pzc
        ab = jax.lax.dot_general(cent, pc, (((1,), (0,)), ((), ())),
                                 preferred_element_type=jnp.float32)
        sqr = (a2 + b2) - 2.0 * ab
        maskf = jnp.where(sqr <= RAD2, 1.0, 0.0)
        local = _dot(maskf, tri, (((1,), (0,))))
        rank = base + local
        conts = [jnp.sum(jnp.where(rank <= float(j), 1.0, 0.0), axis=1,
                         keepdims=True) for j in range(K)]
        acc = acc + jnp.concatenate(conts, axis=1)
        base = base + jnp.sum(maskf, axis=1, keepdims=True)
        return acc, base

    acc0 = jnp.zeros((NFPS, K), jnp.float32)
    base0 = jnp.zeros((NFPS, 1), jnp.float32)
    acc, _ = jax.lax.fori_loop(0, BQC, body, (acc0, base0))
    first = acc[:, 0:1]
    filled = jnp.where(acc == float(N), jnp.broadcast_to(first, acc.shape),
                       acc)
    local = jnp.clip(filled, 0.0, float(N - 1)).astype(jnp.int32)
    idx_ref[0] = local + pl.program_id(0) * N


def _run_bq(cent, cr2, tri):
    return pl.pallas_call(
        _bq_kernel,
        grid=(B,),
        in_specs=[
            pl.BlockSpec((1, NFPS, 3), lambda b: (b, 0, 0)),
            pl.BlockSpec((1, 3, BQC, BQW), lambda b: (b, 0, 0, 0)),
            pl.BlockSpec((BQW, BQW), lambda b: (0, 0)),
        ],
        out_specs=pl.BlockSpec((1, NFPS, K), lambda b: (b, 0, 0)),
        out_shape=jax.ShapeDtypeStruct((B, NFPS, K), jnp.int32),
        compiler_params=pltpu.CompilerParams(
            dimension_semantics=("parallel",)),
    )(cent, cr2, tri)


NW = 32
RPW = M_TOT // NW
CH = 128
NCH = RPW // CH


def _sc_gather_body(tab_ref, idx_ref, out_ref, idx_v, buf, sem):
    wid = lax.axis_index("s") * 2 + lax.axis_index("c")
    base = wid * RPW
    pltpu.sync_copy(idx_ref.at[pl.ds(base, RPW)], idx_v)

    def body(j, carry):
        off = pl.multiple_of(j * CH, CH)
        pltpu.async_copy(tab_ref.at[idx_v.at[pl.ds(off, CH)]], buf,
                         sem).wait()
        pltpu.sync_copy(buf, out_ref.at[pl.ds(base + off, CH)])
        return carry

    lax.fori_loop(0, NCH, body, 0)


def _run_sc_gather(table_all, idx_flat):
    mesh = plsc.VectorSubcoreMesh(core_axis_name="c", subcore_axis_name="s")
    k = functools.partial(
        pl.kernel,
        mesh=mesh,
        compiler_params=pltpu.CompilerParams(use_tc_tiling_on_sc=False),
        out_type=jax.ShapeDtypeStruct((M_TOT, DPAD), jnp.float32),
        scratch_types=[
            pltpu.VMEM((RPW,), jnp.int32),
            pltpu.VMEM((CH, DPAD), jnp.float32),
            pltpu.SemaphoreType.DMA,
        ],
    )(_sc_gather_body)
    return k(table_all, idx_flat)


def _gram_kernel(x_ref, c_ref, g_ref):
    x = x_ref[...]
    xc = jnp.concatenate([x[:, 0:3] - c_ref[...], x[:, 3:]], axis=1)
    xa = jnp.concatenate([xc, jnp.ones((TL, 1), jnp.float32)], axis=1)
    g = _dot(xa, xa, (((0,), (0,))), precision=None)

    @pl.when(pl.program_id(0) == 0)
    def _():
        g_ref[...] = jnp.zeros_like(g_ref)

    g_ref[...] += g


def _run_gram(x, crepf):
    steps = M_TOT // TL
    return pl.pallas_call(
        _gram_kernel,
        grid=(steps,),
        in_specs=[
            pl.BlockSpec((TL, DPAD), lambda s: (s, 0)),
            pl.BlockSpec((TL, 3), lambda s: (s, 0)),
        ],
        out_specs=pl.BlockSpec((DPAD + 1, DPAD + 1), lambda s: (0, 0)),
        out_shape=jax.ShapeDtypeStruct((DPAD + 1, DPAD + 1), jnp.float32),
        compiler_params=pltpu.CompilerParams(
            dimension_semantics=("arbitrary",)),
    )(x, crepf)


def _stats_kernel(g_ref, wt_ref, ga_ref, be_ref, scale_ref, bias_ref, *,
                  din):
    G = g_ref[...]
    m = G[din:din + 1, 0:din] / M_TOT
    gx = G[0:din, 0:din] / M_TOT
    cov = gx - _dot(m, m, (((0,), (0,))))
    wt = wt_ref[...]
    mean_y = _dot(m, wt, (((1,), (0,))))
    covw = _dot(cov, wt, (((1,), (0,))))
    var_y = jnp.sum(wt * covw, axis=0, keepdims=True)
    scale = ga_ref[...] / jnp.sqrt(var_y + EPSBN)
    scale_ref[...] = scale
    bias_ref[...] = be_ref[...] - scale * mean_y


def _run_stats(g, wt, ga, be):
    din = wt.shape[0]
    dout = wt.shape[1]
    return pl.pallas_call(
        functools.partial(_stats_kernel, din=din),
        out_shape=[
            jax.ShapeDtypeStruct((1, dout), jnp.float32),
            jax.ShapeDtypeStruct((1, dout), jnp.float32),
        ],
    )(g, wt, ga, be)


def _layer1_kernel(x_ref, c_ref, wt_ref, s_ref, b_ref, v_ref, g_ref):
    x = x_ref[...]
    xc = jnp.concatenate([x[:, 0:3] - c_ref[...], x[:, 3:]], axis=1)
    y = _dot(xc, wt_ref[...], (((1,), (0,))))
    v = jnp.maximum(y * s_ref[...] + b_ref[...], 0.0)
    v_ref[...] = v
    va = jnp.concatenate([v, jnp.ones((TL, 1), jnp.float32)], axis=1)
    g = _dot(va, va, (((0,), (0,))), precision=None)

    @pl.when(pl.program_id(0) == 0)
    def _():
        g_ref[...] = jnp.zeros_like(g_ref)

    g_ref[...] += g


def _run_layer1(x, crepf, wt, scale, bias):
    steps = M_TOT // TL
    din = wt.shape[0]
    dout = wt.shape[1]
    return pl.pallas_call(
        _layer1_kernel,
        grid=(steps,),
        in_specs=[
            pl.BlockSpec((TL, din), lambda s: (s, 0)),
            pl.BlockSpec((TL, 3), lambda s: (s, 0)),
            pl.BlockSpec((din, dout), lambda s: (0, 0)),
            pl.BlockSpec((1, dout), lambda s: (0, 0)),
            pl.BlockSpec((1, dout), lambda s: (0, 0)),
        ],
        out_specs=[
            pl.BlockSpec((TL, dout), lambda s: (s, 0)),
            pl.BlockSpec((dout + 1, dout + 1), lambda s: (0, 0)),
        ],
        out_shape=[
            jax.ShapeDtypeStruct((M_TOT, dout), jnp.float32),
            jax.ShapeDtypeStruct((dout + 1, dout + 1), jnp.float32),
        ],
        compiler_params=pltpu.CompilerParams(
            dimension_semantics=("arbitrary",)),
    )(x, crepf, wt, scale, bias)


def _layer_kernel(x_ref, wt_ref, s_ref, b_ref, v_ref, g_ref):
    y = _dot(x_ref[...], wt_ref[...], (((1,), (0,))))
    v = jnp.maximum(y * s_ref[...] + b_ref[...], 0.0)
    v_ref[...] = v
    va = jnp.concatenate([v, jnp.ones((TL, 1), jnp.float32)], axis=1)
    g = _dot(va, va, (((0,), (0,))), precision=None)

    @pl.when(pl.program_id(0) == 0)
    def _():
        g_ref[...] = jnp.zeros_like(g_ref)

    g_ref[...] += g


def _run_layer(x, wt, scale, bias):
    steps = M_TOT // TL
    din = wt.shape[0]
    dout = wt.shape[1]
    return pl.pallas_call(
        _layer_kernel,
        grid=(steps,),
        in_specs=[
            pl.BlockSpec((TL, din), lambda s: (s, 0)),
            pl.BlockSpec((din, dout), lambda s: (0, 0)),
            pl.BlockSpec((1, dout), lambda s: (0, 0)),
            pl.BlockSpec((1, dout), lambda s: (0, 0)),
        ],
        out_specs=[
            pl.BlockSpec((TL, dout), lambda s: (s, 0)),
            pl.BlockSpec((dout + 1, dout + 1), lambda s: (0, 0)),
        ],
        out_shape=[
            jax.ShapeDtypeStruct((M_TOT, dout), jnp.float32),
            jax.ShapeDtypeStruct((dout + 1, dout + 1), jnp.float32),
        ],
        compiler_params=pltpu.CompilerParams(
            dimension_semantics=("arbitrary",)),
    )(x, wt, scale, bias)


def _layer3_kernel(x_ref, wt_ref, s_ref, b_ref, o_ref):
    y = _dot(x_ref[...], wt_ref[...], (((1,), (0,))))
    v = jnp.maximum(y * s_ref[...] + b_ref[...], 0.0)
    for g in range(TL // K):
        o_ref[g:g + 1, :] = jnp.max(v[g * K:(g + 1) * K, :], axis=0,
                                    keepdims=True)


def _run_layer3(x, wt, scale, bias):
    steps = M_TOT // TL
    din = wt.shape[0]
    dout = wt.shape[1]
    return pl.pallas_call(
        _layer3_kernel,
        grid=(steps,),
        in_specs=[
            pl.BlockSpec((TL, din), lambda s: (s, 0)),
            pl.BlockSpec((din, dout), lambda s: (0, 0)),
            pl.BlockSpec((1, dout), lambda s: (0, 0)),
            pl.BlockSpec((1, dout), lambda s: (0, 0)),
        ],
        out_specs=pl.BlockSpec((TL // K, dout), lambda s: (s, 0)),
        out_shape=jax.ShapeDtypeStruct((B * NFPS, dout), jnp.float32),
        compiler_params=pltpu.CompilerParams(
            dimension_semantics=("parallel",)),
    )(x, wt, scale, bias)


def kernel(coords, features, W1, b1, g1, be1, W2, b2, g2, be2,
           W3, b3, g3, be3):
    cr = coords.reshape(B, 3, NSUB, NLANE)
    cent = _run_fps(cr)
    new_coords = cent.transpose(0, 2, 1)

    tri = jnp.triu(jnp.ones((BQW, BQW), jnp.float32))
    idx = _run_bq(cent, coords.reshape(B, 3, BQC, BQW), tri)

    table_all = jnp.concatenate(
        [coords.transpose(0, 2, 1), features.transpose(0, 2, 1),
         jnp.zeros((B, N, DPAD - 3 - CF), jnp.float32)],
        axis=2).reshape(B * N, DPAD)
    x = _run_sc_gather(table_all, idx.reshape(-1))
    crepf = jnp.repeat(cent, K, axis=1).reshape(M_TOT, 3)
    g0 = _run_gram(x, crepf)

    w1t = jnp.concatenate(
        [W1, jnp.zeros((W1.shape[0], DPAD - W1.shape[1]), jnp.float32)],
        axis=1).T
    s1, bi1 = _run_stats(g0, w1t, g1.reshape(1, -1), be1.reshape(1, -1))
    v1, gm1 = _run_layer1(x, crepf, w1t, s1, bi1)

    w2t = W2.T
    s2, bi2 = _run_stats(gm1, w2t, g2.reshape(1, -1), be2.reshape(1, -1))
    v2, gm2 = _run_layer(v1, w2t, s2, bi2)

    w3t = W3.T
    s3, bi3 = _run_stats(gm2, w3t, g3.reshape(1, -1), be3.reshape(1, -1))
    pooled = _run_layer3(v2, w3t, s3, bi3)

    new_features = pooled.reshape(B, NFPS, -1).transpose(0, 2, 1)
    return (new_coords, new_features)
